# baseline (speedup 1.0000x reference)
import functools

import jax
import jax.numpy as jnp
from jax import lax
from jax.experimental import pallas as pl
from jax.experimental.pallas import tpu as pltpu
from jax.experimental.pallas import tpu_sc as plsc


def _make_segment_sum_sc(n, d, e):
    info = plsc.get_sparse_core_info()
    nc, ns = info.num_cores, info.num_subcores
    nw = nc * ns

    chunk = 40
    edges_per_worker = e // nw
    nchunks = edges_per_worker // chunk
    assert edges_per_worker * nw == e and nchunks * chunk == edges_per_worker

    rows_per_tile = (n // ns) // 8 * 8
    rows_rem = n - rows_per_tile * ns
    assert rows_rem % 8 == 0

    nbuf = 5
    assert nchunks % nbuf == 0
    nouter = nchunks // nbuf

    mesh = plsc.VectorSubcoreMesh(core_axis_name="c", subcore_axis_name="s")

    @functools.partial(
        pl.kernel,
        mesh=mesh,
        out_type=jax.ShapeDtypeStruct((nc, n, d), jnp.float32),
        scratch_types=(
            [pltpu.VMEM((edges_per_worker,), jnp.int32)]
            + [pltpu.VMEM((chunk,), jnp.int32) for _ in range(nbuf)]
            + [pltpu.VMEM((chunk, d), jnp.float32) for _ in range(nbuf)]
            + [pltpu.VMEM_SHARED((n, d), jnp.float32)]
            + [pltpu.SemaphoreType.DMA for _ in range(3 * nbuf + 1)]
        ),
    )
    def seg_sum(h_hbm, src_hbm, dst_hbm, out_hbm, *scratch):
        src_all = scratch[0]
        dst_ring = scratch[1:1 + nbuf]
        rows_ring = scratch[1 + nbuf:1 + 2 * nbuf]
        acc_sh = scratch[1 + 2 * nbuf]
        sem_d = scratch[2 + 2 * nbuf:2 + 3 * nbuf]
        sem_g = scratch[2 + 3 * nbuf:2 + 4 * nbuf]
        sem_s = scratch[2 + 4 * nbuf:2 + 5 * nbuf]
        sem_z = scratch[2 + 5 * nbuf]

        cid = lax.axis_index("c")
        sid = lax.axis_index("s")
        wid = cid * ns + sid
        ebase = wid * edges_per_worker

        pltpu.async_copy(src_hbm.at[pl.ds(ebase, edges_per_worker)], src_all,
                         sem_z)

        rows_v = rows_ring[0]
        zero16 = jnp.zeros((16,), jnp.float32)

        def _zero_row(r, _):
            for j in range(d // 16):
                rows_v[r, pl.ds(j * 16, 16)] = zero16
            return 0

        lax.fori_loop(0, chunk, _zero_row, 0)
        pltpu.make_async_copy(src_hbm.at[pl.ds(ebase, edges_per_worker)],
                              src_all, sem_z).wait()

        row0 = sid * rows_per_tile

        def _row_chunks(count):
            done = 0
            while done < count:
                step = min(chunk, count - done)
                yield done, step
                done += step

        zcopies = []
        for off, step in _row_chunks(rows_per_tile):
            zcopies.append(pltpu.async_copy(
                rows_v.at[pl.ds(0, step)],
                acc_sh.at[pl.ds(row0 + off, step)], sem_z))

        @pl.when(sid == ns - 1)
        def _zero_tail():
            tail = []
            for off, step in _row_chunks(rows_rem):
                tail.append(pltpu.async_copy(
                    rows_v.at[pl.ds(0, step)],
                    acc_sh.at[pl.ds(ns * rows_per_tile + off, step)], sem_z))
            for cp in tail:
                cp.wait()

        for cp in zcopies:
            cp.wait()

        plsc.subcore_barrier()

        lag = 2

        def _issue(j, k):
            pltpu.async_copy(dst_hbm.at[pl.ds(ebase + j * chunk, chunk)],
                             dst_ring[k], sem_d[k])
            pltpu.async_copy(h_hbm.at[src_all.at[pl.ds(j * chunk, chunk)]],
                             rows_ring[k], sem_g[k])

        def _wait(j, k):
            pltpu.make_async_copy(
                dst_hbm.at[pl.ds(ebase + j * chunk, chunk)],
                dst_ring[k], sem_d[k]).wait()
            pltpu.make_async_copy(
                h_hbm.at[src_all.at[pl.ds(j * chunk, chunk)]],
                rows_ring[k], sem_g[k]).wait()

        def _wait_scatter(k):
            pltpu.make_async_copy(rows_ring[k], acc_sh.at[dst_ring[k]],
                                  sem_s[k]).wait()

        for k in range(nbuf):
            _issue(k, k)

        def _outer(i, _):
            for k in range(nbuf):
                j = i * nbuf + k
                _wait(j, k)
                pltpu.async_copy(rows_ring[k], acc_sh.at[dst_ring[k]],
                                 sem_s[k], add=True)
                jl = j - lag
                kl = (k - lag) % nbuf

                @pl.when((jl >= 0) & (jl + nbuf < nchunks))
                def _drain_and_reissue():
                    _wait_scatter(kl)
                    _issue(jl + nbuf, kl)
            return 0

        lax.fori_loop(0, nouter, _outer, 0)
        for k in range(nbuf):
            _wait_scatter(k)

        plsc.subcore_barrier()

        pltpu.sync_copy(acc_sh.at[pl.ds(row0, rows_per_tile)],
                        out_hbm.at[cid, pl.ds(row0, rows_per_tile)])

        @pl.when(sid == ns - 1)
        def _write_tail():
            base = ns * rows_per_tile
            pltpu.sync_copy(acc_sh.at[pl.ds(base, rows_rem)],
                            out_hbm.at[cid, pl.ds(base, rows_rem)])

    return seg_sum


def _layer_body(eps_ref, h_ref, a0_ref, a1_ref, w1_ref, b1_ref, w2_ref,
                b2_ref, g_ref, be_ref, out_ref):
    z = h_ref[...] * (1.0 + eps_ref[0]) + a0_ref[...] + a1_ref[...]
    z = jnp.dot(z, w1_ref[...], preferred_element_type=jnp.float32)
    z = jnp.maximum(z + b1_ref[...], 0.0)
    z = jnp.dot(z, w2_ref[...], preferred_element_type=jnp.float32)
    z = jnp.maximum(z + b2_ref[...], 0.0)
    mu = jnp.mean(z, axis=0, keepdims=True)
    var = jnp.mean(jnp.square(z - mu), axis=0, keepdims=True)
    out_ref[...] = ((z - mu) * lax.rsqrt(var + 1e-5) * g_ref[...]
                    + be_ref[...])


def _head_body(h_ref, batch_ref, l1w_ref, l1b_ref, l2w_ref, l2b_ref, out_ref):
    n = h_ref.shape[0]
    g = out_ref.shape[0]
    b = batch_ref[...]
    gids = lax.broadcasted_iota(jnp.int32, (n, g), 1)
    onehot = (b == gids).astype(jnp.float32)
    sums = lax.dot_general(onehot, h_ref[...], (((0,), (0,)), ((), ())),
                           preferred_element_type=jnp.float32)
    counts = jnp.sum(onehot, axis=0)[:, None]
    pooled = sums / jnp.maximum(counts, 1.0)
    y = jnp.dot(pooled, l1w_ref[...], preferred_element_type=jnp.float32)
    y = jnp.maximum(y + l1b_ref[...], 0.0)
    y = jnp.dot(y, l2w_ref[...], preferred_element_type=jnp.float32)
    y = y + l2b_ref[...]
    m = jnp.max(y, axis=-1, keepdims=True)
    lse = jnp.log(jnp.sum(jnp.exp(y - m), axis=-1, keepdims=True)) + m
    out_ref[...] = y - lse


def kernel(x, edge_index, batch, W1, b1, W2, b2, gamma, beta, eps,
           lin1_W, lin1_b, lin2_W, lin2_b):
    n, d = x.shape
    e = edge_index.shape[1]
    h_dim = W1.shape[2]
    out_dim = lin2_W.shape[1]
    num_layers = W1.shape[0]
    g = 64

    src = edge_index[0]
    dst = edge_index[1]

    seg_sum = _make_segment_sum_sc(n, d, e)

    layer_call = pl.pallas_call(
        _layer_body,
        out_shape=jax.ShapeDtypeStruct((n, h_dim), jnp.float32),
        in_specs=[pl.BlockSpec(memory_space=pltpu.SMEM)] + [pl.BlockSpec()] * 9,
        out_specs=pl.BlockSpec(),
    )

    head_call = pl.pallas_call(
        _head_body,
        out_shape=jax.ShapeDtypeStruct((g, out_dim), jnp.float32),
    )

    h = x
    for i in range(num_layers):
        agg = seg_sum(h, src, dst)
        h = layer_call(
            jnp.reshape(1.0 * eps[i], (1,)),
            h, agg[0], agg[1],
            W1[i], jnp.reshape(b1[i], (1, h_dim)),
            W2[i], jnp.reshape(b2[i], (1, h_dim)),
            jnp.reshape(gamma[i], (1, h_dim)),
            jnp.reshape(beta[i], (1, h_dim)),
        )

    out = head_call(
        h,
        jnp.reshape(batch, (n, 1)),
        lin1_W, jnp.reshape(lin1_b, (1, h_dim)),
        lin2_W, jnp.reshape(lin2_b, (1, out_dim)),
    )
    return out


# speedup vs baseline: 1.1243x; 1.1243x over previous
import functools

import jax
import jax.numpy as jnp
from jax import lax
from jax.experimental import pallas as pl
from jax.experimental.pallas import tpu as pltpu
from jax.experimental.pallas import tpu_sc as plsc


def _make_segment_sum_sc(n, d, e):
    info = plsc.get_sparse_core_info()
    nc, ns = info.num_cores, info.num_subcores
    nw = nc * ns

    chunk = 40
    edges_per_worker = e // nw
    nchunks = edges_per_worker // chunk
    assert edges_per_worker * nw == e and nchunks * chunk == edges_per_worker

    rows_per_tile = (n // ns) // 8 * 8
    rows_rem = n - rows_per_tile * ns
    assert rows_rem % 8 == 0

    nbuf = 5
    assert nchunks % nbuf == 0
    nouter = nchunks // nbuf

    mesh = plsc.VectorSubcoreMesh(core_axis_name="c", subcore_axis_name="s")

    @functools.partial(
        pl.kernel,
        mesh=mesh,
        out_type=jax.ShapeDtypeStruct((nc, n, d), jnp.float32),
        scratch_types=(
            [pltpu.VMEM((edges_per_worker,), jnp.int32)]
            + [pltpu.VMEM((chunk,), jnp.int32) for _ in range(nbuf)]
            + [pltpu.VMEM((chunk, d), jnp.float32) for _ in range(nbuf)]
            + [pltpu.VMEM_SHARED((n, d), jnp.float32)]
            + [pltpu.SemaphoreType.DMA for _ in range(2 * nbuf + 1)]
        ),
    )
    def seg_sum(h_hbm, src_hbm, dst_hbm, out_hbm, *scratch):
        src_all = scratch[0]
        dst_ring = scratch[1:1 + nbuf]
        rows_ring = scratch[1 + nbuf:1 + 2 * nbuf]
        acc_sh = scratch[1 + 2 * nbuf]
        sem_d = scratch[2 + 2 * nbuf:2 + 3 * nbuf]
        sem_g = scratch[2 + 3 * nbuf:2 + 4 * nbuf]
        sem_z = scratch[2 + 4 * nbuf]

        cid = lax.axis_index("c")
        sid = lax.axis_index("s")
        wid = cid * ns + sid
        ebase = wid * edges_per_worker

        pltpu.async_copy(src_hbm.at[pl.ds(ebase, edges_per_worker)], src_all,
                         sem_z)

        rows_v = rows_ring[0]
        zero16 = jnp.zeros((16,), jnp.float32)

        def _zero_row(r, _):
            for j in range(d // 16):
                rows_v[r, pl.ds(j * 16, 16)] = zero16
            return 0

        lax.fori_loop(0, chunk, _zero_row, 0)
        pltpu.make_async_copy(src_hbm.at[pl.ds(ebase, edges_per_worker)],
                              src_all, sem_z).wait()

        row0 = sid * rows_per_tile

        def _row_chunks(count):
            done = 0
            while done < count:
                step = min(chunk, count - done)
                yield done, step
                done += step

        zcopies = []
        for off, step in _row_chunks(rows_per_tile):
            zcopies.append(pltpu.async_copy(
                rows_v.at[pl.ds(0, step)],
                acc_sh.at[pl.ds(row0 + off, step)], sem_z))

        @pl.when(sid == ns - 1)
        def _zero_tail():
            tail = []
            for off, step in _row_chunks(rows_rem):
                tail.append(pltpu.async_copy(
                    rows_v.at[pl.ds(0, step)],
                    acc_sh.at[pl.ds(ns * rows_per_tile + off, step)], sem_z))
            for cp in tail:
                cp.wait()

        for cp in zcopies:
            cp.wait()

        plsc.subcore_barrier()

        def _issue(j, k):
            pltpu.async_copy(dst_hbm.at[pl.ds(ebase + j * chunk, chunk)],
                             dst_ring[k], sem_d[k])
            pltpu.async_copy(h_hbm.at[src_all.at[pl.ds(j * chunk, chunk)]],
                             rows_ring[k], sem_g[k])

        def _wait(j, k):
            pltpu.make_async_copy(
                dst_hbm.at[pl.ds(ebase + j * chunk, chunk)],
                dst_ring[k], sem_d[k]).wait()
            pltpu.make_async_copy(
                h_hbm.at[src_all.at[pl.ds(j * chunk, chunk)]],
                rows_ring[k], sem_g[k]).wait()

        for k in range(nbuf):
            _issue(k, k)

        def _outer(i, _):
            for k in range(nbuf):
                j = i * nbuf + k
                _wait(j, k)
                pltpu.sync_copy(rows_ring[k], acc_sh.at[dst_ring[k]], add=True)
                jn = j + nbuf

                @pl.when(jn < nchunks)
                def _reissue():
                    _issue(jn, k)
            return 0

        lax.fori_loop(0, nouter, _outer, 0)

        plsc.subcore_barrier()

        pltpu.sync_copy(acc_sh.at[pl.ds(row0, rows_per_tile)],
                        out_hbm.at[cid, pl.ds(row0, rows_per_tile)])

        @pl.when(sid == ns - 1)
        def _write_tail():
            base = ns * rows_per_tile
            pltpu.sync_copy(acc_sh.at[pl.ds(base, rows_rem)],
                            out_hbm.at[cid, pl.ds(base, rows_rem)])

    return seg_sum


def _layer_body(eps_ref, h_ref, a0_ref, a1_ref, w1_ref, b1_ref, w2_ref,
                b2_ref, g_ref, be_ref, out_ref):
    z = h_ref[...] * (1.0 + eps_ref[0]) + a0_ref[...] + a1_ref[...]
    z = jnp.dot(z, w1_ref[...], preferred_element_type=jnp.float32)
    z = jnp.maximum(z + b1_ref[...], 0.0)
    z = jnp.dot(z, w2_ref[...], preferred_element_type=jnp.float32)
    z = jnp.maximum(z + b2_ref[...], 0.0)
    mu = jnp.mean(z, axis=0, keepdims=True)
    var = jnp.mean(jnp.square(z - mu), axis=0, keepdims=True)
    out_ref[...] = ((z - mu) * lax.rsqrt(var + 1e-5) * g_ref[...]
                    + be_ref[...])


def _head_body(h_ref, batch_ref, l1w_ref, l1b_ref, l2w_ref, l2b_ref, out_ref):
    n = h_ref.shape[0]
    g = out_ref.shape[0]
    b = batch_ref[...]
    gids = lax.broadcasted_iota(jnp.int32, (n, g), 1)
    onehot = (b == gids).astype(jnp.float32)
    sums = lax.dot_general(onehot, h_ref[...], (((0,), (0,)), ((), ())),
                           preferred_element_type=jnp.float32)
    counts = jnp.sum(onehot, axis=0)[:, None]
    pooled = sums / jnp.maximum(counts, 1.0)
    y = jnp.dot(pooled, l1w_ref[...], preferred_element_type=jnp.float32)
    y = jnp.maximum(y + l1b_ref[...], 0.0)
    y = jnp.dot(y, l2w_ref[...], preferred_element_type=jnp.float32)
    y = y + l2b_ref[...]
    m = jnp.max(y, axis=-1, keepdims=True)
    lse = jnp.log(jnp.sum(jnp.exp(y - m), axis=-1, keepdims=True)) + m
    out_ref[...] = y - lse


def kernel(x, edge_index, batch, W1, b1, W2, b2, gamma, beta, eps,
           lin1_W, lin1_b, lin2_W, lin2_b):
    n, d = x.shape
    e = edge_index.shape[1]
    h_dim = W1.shape[2]
    out_dim = lin2_W.shape[1]
    num_layers = W1.shape[0]
    g = 64

    src = edge_index[0]
    dst = edge_index[1]

    seg_sum = _make_segment_sum_sc(n, d, e)

    layer_call = pl.pallas_call(
        _layer_body,
        out_shape=jax.ShapeDtypeStruct((n, h_dim), jnp.float32),
        in_specs=[pl.BlockSpec(memory_space=pltpu.SMEM)] + [pl.BlockSpec()] * 9,
        out_specs=pl.BlockSpec(),
    )

    head_call = pl.pallas_call(
        _head_body,
        out_shape=jax.ShapeDtypeStruct((g, out_dim), jnp.float32),
    )

    h = x
    for i in range(num_layers):
        agg = seg_sum(h, src, dst)
        h = layer_call(
            jnp.reshape(1.0 * eps[i], (1,)),
            h, agg[0], agg[1],
            W1[i], jnp.reshape(b1[i], (1, h_dim)),
            W2[i], jnp.reshape(b2[i], (1, h_dim)),
            jnp.reshape(gamma[i], (1, h_dim)),
            jnp.reshape(beta[i], (1, h_dim)),
        )

    out = head_call(
        h,
        jnp.reshape(batch, (n, 1)),
        lin1_W, jnp.reshape(lin1_b, (1, h_dim)),
        lin2_W, jnp.reshape(lin2_b, (1, out_dim)),
    )
    return out
